# baseline (speedup 1.0000x reference)
import functools

import jax
import jax.numpy as jnp
from jax import lax
from jax.experimental import pallas as pl
from jax.experimental.pallas import tpu as pltpu, tpu_sc as plsc

BATCH = 16384
EMB_DIM = 64
NUM_WORKERS = 32
B_PER_W = BATCH // NUM_WORKERS


def _transe_gather(h, r, t, ent_emb, rel_emb):
    mesh = plsc.VectorSubcoreMesh(core_axis_name="c", subcore_axis_name="s")

    @functools.partial(
        pl.kernel,
        mesh=mesh,
        compiler_params=pltpu.CompilerParams(use_tc_tiling_on_sc=False),
        out_type=[
            jax.ShapeDtypeStruct((BATCH, EMB_DIM), jnp.float32),
            jax.ShapeDtypeStruct((BATCH, EMB_DIM), jnp.float32),
            jax.ShapeDtypeStruct((BATCH, EMB_DIM), jnp.float32),
        ],
        scratch_types=[
            pltpu.VMEM((B_PER_W,), jnp.int32),
            pltpu.VMEM((B_PER_W,), jnp.int32),
            pltpu.VMEM((B_PER_W,), jnp.int32),
            pltpu.VMEM((B_PER_W, EMB_DIM), jnp.float32),
            pltpu.VMEM((B_PER_W, EMB_DIM), jnp.float32),
            pltpu.VMEM((B_PER_W, EMB_DIM), jnp.float32),
            pltpu.SemaphoreType.DMA,
        ],
    )
    def k(h_hbm, r_hbm, t_hbm, ent_hbm, rel_hbm,
          h_out, r_out, t_out,
          hi_v, ri_v, ti_v, hrows, rrows, trows, sem):
        wid = lax.axis_index("s") * 2 + lax.axis_index("c")
        sl = pl.ds(wid * B_PER_W, B_PER_W)
        pltpu.sync_copy(h_hbm.at[sl], hi_v)
        pltpu.sync_copy(r_hbm.at[sl], ri_v)
        pltpu.sync_copy(t_hbm.at[sl], ti_v)
        ch = pltpu.async_copy(ent_hbm.at[hi_v], hrows, sem)
        cr = pltpu.async_copy(rel_hbm.at[ri_v], rrows, sem)
        ct = pltpu.async_copy(ent_hbm.at[ti_v], trows, sem)
        ch.wait()
        pltpu.sync_copy(hrows, h_out.at[sl])
        cr.wait()
        pltpu.sync_copy(rrows, r_out.at[sl])
        ct.wait()
        pltpu.sync_copy(trows, t_out.at[sl])

    return k(h, r, t, ent_emb, rel_emb)


def kernel(h, r, t, ent_emb, rel_emb):
    h = h.astype(jnp.int32)
    r = r.astype(jnp.int32)
    t = t.astype(jnp.int32)
    h_e, r_e, t_e = _transe_gather(h, r, t, ent_emb, rel_emb)
    return (h_e, r_e, t_e)


# speedup vs baseline: 1.6630x; 1.6630x over previous
import functools

import jax
import jax.numpy as jnp
from jax import lax
from jax.experimental import pallas as pl
from jax.experimental.pallas import tpu as pltpu, tpu_sc as plsc

BATCH = 16384
EMB_DIM = 64
NUM_WORKERS = 32
B_PER_W = BATCH // NUM_WORKERS
CHUNK = 128
N_CHUNKS = B_PER_W // CHUNK


def _transe_gather(h, r, t, ent_emb, rel_emb):
    mesh = plsc.VectorSubcoreMesh(core_axis_name="c", subcore_axis_name="s")

    row_buf = lambda: pltpu.VMEM((CHUNK, EMB_DIM), jnp.float32)
    idx_buf = lambda: pltpu.SMEM((CHUNK,), jnp.int32)

    @functools.partial(
        pl.kernel,
        mesh=mesh,
        compiler_params=pltpu.CompilerParams(use_tc_tiling_on_sc=True),
        out_type=[
            jax.ShapeDtypeStruct((BATCH, EMB_DIM), jnp.float32),
            jax.ShapeDtypeStruct((BATCH, EMB_DIM), jnp.float32),
            jax.ShapeDtypeStruct((BATCH, EMB_DIM), jnp.float32),
        ],
        scratch_types=[
            [[row_buf(), row_buf(), row_buf()] for _ in range(2)],
            [idx_buf(), idx_buf(), idx_buf()],
            [pltpu.VMEM((B_PER_W,), jnp.int32) for _ in range(3)],
            pltpu.SemaphoreType.DMA,
            pltpu.SemaphoreType.DMA,
        ],
    )
    def k(h_hbm, r_hbm, t_hbm, ent_hbm, rel_hbm,
          h_out, r_out, t_out,
          rows, idxs, vidxs, gsem, wsem):
        wid = lax.axis_index("s") * 2 + lax.axis_index("c")
        base = wid * B_PER_W
        idx_srcs = (h_hbm, r_hbm, t_hbm)
        tabs = (ent_hbm, rel_hbm, ent_hbm)
        outs = (h_out, r_out, t_out)

        for j in range(3):
            pltpu.sync_copy(idx_srcs[j].at[pl.ds(base, B_PER_W)], vidxs[j])

        for c in range(N_CHUNKS):
            b = c % 2
            off = base + c * CHUNK
            csl = pl.ds(off, CHUNK)
            if c >= 2:
                for j in range(3):
                    pltpu.make_async_copy(rows[b][j], outs[j].at[csl], wsem).wait()

            def fire(k, _):
                for j in range(3):
                    vec = vidxs[j][pl.ds(c * CHUNK + k * 16, 16)]
                    for lane in range(16):
                        pltpu.async_copy(tabs[j].at[vec[lane]],
                                         rows[b][j].at[k * 16 + lane], gsem)
                return 0
            lax.fori_loop(0, CHUNK // 16, fire, 0)

            def drain(i, _):
                for j in range(3):
                    pltpu.make_async_copy(tabs[j].at[0], rows[b][j].at[0], gsem).wait()
                return 0
            lax.fori_loop(0, CHUNK, drain, 0)

            for j in range(3):
                pltpu.async_copy(rows[b][j], outs[j].at[csl], wsem)

        for c in range(max(0, N_CHUNKS - 2), N_CHUNKS):
            b = c % 2
            csl = pl.ds(base + c * CHUNK, CHUNK)
            for j in range(3):
                pltpu.make_async_copy(rows[b][j], outs[j].at[csl], wsem).wait()

    return k(h, r, t, ent_emb, rel_emb)


def kernel(h, r, t, ent_emb, rel_emb):
    h = h.astype(jnp.int32)
    r = r.astype(jnp.int32)
    t = t.astype(jnp.int32)
    h_e, r_e, t_e = _transe_gather(h, r, t, ent_emb, rel_emb)
    return (h_e, r_e, t_e)
